# baseline (speedup 1.0000x reference)
import functools

import jax
import jax.numpy as jnp
from jax import lax
from jax.experimental import pallas as pl
from jax.experimental.pallas import tpu as pltpu
from jax.experimental.pallas import tpu_sc as plsc

NA = 10000
NP = 10000
E = 320000
D = 128
H = 64
OUT = 4

NC = 2
NS = 16
NR = 10240
RPT = NR // NS
CHUNK = 128
CH_B = 160
CH_D = CH_B // 2
BM = 1000
GRID = NA // BM


def _pack_edges(ei):
    tot = NS * CH_B * CHUNK
    padn = tot - E
    src = jnp.concatenate(
        [ei[0].astype(jnp.int32), jnp.zeros((padn,), jnp.int32)])
    dump = NA + 48 + (jnp.arange(padn, dtype=jnp.int32) % 128)
    dst = jnp.concatenate([ei[1].astype(jnp.int32), dump])
    return (src.reshape(NS, CH_B, CHUNK), dst.reshape(NS, CH_B, CHUNK))


def _tab_body(xa_ref, xp_ref, wa_ref, wp_ref, out_ref):
    out_ref[0] = jnp.dot(xa_ref[...], wa_ref[...],
                         preferred_element_type=jnp.float32)
    out_ref[1] = jnp.dot(xp_ref[...], wp_ref[...],
                         preferred_element_type=jnp.float32)


def _make_tables(xa, xp, wa, wp):
    return pl.pallas_call(
        _tab_body,
        grid=(GRID,),
        in_specs=[
            pl.BlockSpec((BM, D), lambda i: (i, 0)),
            pl.BlockSpec((BM, D), lambda i: (i, 0)),
            pl.BlockSpec((D, H), lambda i: (0, 0)),
            pl.BlockSpec((D, H), lambda i: (0, 0)),
        ],
        out_specs=pl.BlockSpec((2, BM, H), lambda i: (0, i, 0)),
        out_shape=jax.ShapeDtypeStruct((2, NA, H), jnp.float32),
    )(xa, xp, wa, wp)


def _sc_mesh():
    return plsc.VectorSubcoreMesh(
        core_axis_name="c", subcore_axis_name="s",
        num_cores=NC, num_subcores=NS)


def _sc_agg_l0(tabs, srcs, dsts, z64, z8, ones8):
    @functools.partial(
        pl.kernel,
        out_type=[jax.ShapeDtypeStruct((NC, NR, H), jnp.float32),
                  jax.ShapeDtypeStruct((NC, NR, 8), jnp.float32)],
        mesh=_sc_mesh(),
        compiler_params=pltpu.CompilerParams(use_tc_tiling_on_sc=False),
        scratch_types=[
            pltpu.VMEM((CH_B, CHUNK), jnp.int32),
            pltpu.VMEM((CH_B, CHUNK), jnp.int32),
            pltpu.VMEM((CHUNK, H), jnp.float32),
            pltpu.VMEM((CHUNK, H), jnp.float32),
            pltpu.VMEM((CHUNK, 8), jnp.float32),
            pltpu.VMEM_SHARED((NR, H), jnp.float32),
            pltpu.VMEM_SHARED((NR, 8), jnp.float32),
            pltpu.SemaphoreType.DMA,
            pltpu.SemaphoreType.DMA,
        ],
    )
    def k(tabs_h, srcs_h, dsts_h, z64_h, z8_h, ones8_h, feat_o, cnt_o,
          src_v, dst_v, buf_a, buf_b, ones_v, acc_f, acc_c, sem_a, sem_b):
        c = lax.axis_index("c")
        s = lax.axis_index("s")
        pltpu.sync_copy(srcs_h.at[c, s], src_v)
        pltpu.sync_copy(dsts_h.at[c, s], dst_v)
        pltpu.sync_copy(ones8_h, ones_v)
        r0 = s * RPT
        pltpu.sync_copy(z64_h, acc_f.at[pl.ds(r0, RPT)])
        pltpu.sync_copy(z8_h, acc_c.at[pl.ds(r0, RPT)])
        plsc.subcore_barrier()
        tab = tabs_h.at[c]

        def pair(i, carry):
            g = i * 2
            cp_a = pltpu.async_copy(tab.at[src_v.at[g]], buf_a, sem_a)
            cp_b = pltpu.async_copy(tab.at[src_v.at[g + 1]], buf_b, sem_b)
            cp_a.wait()
            pltpu.sync_copy(buf_a, acc_f.at[dst_v.at[g]], add=True)
            pltpu.sync_copy(ones_v, acc_c.at[dst_v.at[g]], add=True)
            cp_b.wait()
            pltpu.sync_copy(buf_b, acc_f.at[dst_v.at[g + 1]], add=True)
            pltpu.sync_copy(ones_v, acc_c.at[dst_v.at[g + 1]], add=True)
            return carry

        lax.fori_loop(0, CH_B // 2, pair, 0)
        plsc.subcore_barrier()
        pltpu.sync_copy(acc_f.at[pl.ds(r0, RPT)],
                        feat_o.at[c, pl.ds(r0, RPT)])
        pltpu.sync_copy(acc_c.at[pl.ds(r0, RPT)],
                        cnt_o.at[c, pl.ds(r0, RPT)])

    return k(tabs, srcs, dsts, z64, z8, ones8)


def _mid_body(feat_ref, cnt_ref, xp_ref, xa_ref, w0ra_ref, b0a_ref,
              w0rp_ref, b0p_ref, w1l_ref, w1r_ref, b1_ref, wlin_ref,
              blin_ref, g8_ref, base_ref):
    f32 = jnp.float32
    cp = jnp.maximum(cnt_ref[0][:, 0:1], 1.0)
    ca = jnp.maximum(cnt_ref[1][:, 0:1], 1.0)
    hp = (feat_ref[0] / cp + b0a_ref[...][None, :]
          + jnp.dot(xp_ref[...], w0ra_ref[...], preferred_element_type=f32))
    hp = jnp.where(hp >= 0, hp, 0.01 * hp)
    ha = (feat_ref[1] / ca + b0p_ref[...][None, :]
          + jnp.dot(xa_ref[...], w0rp_ref[...], preferred_element_type=f32))
    ha = jnp.where(ha >= 0, ha, 0.01 * ha)
    m1 = jnp.dot(w1l_ref[...], wlin_ref[...], preferred_element_type=f32)
    m1p = jnp.concatenate([m1, jnp.zeros_like(m1)], axis=1)
    m2 = jnp.dot(w1r_ref[...], wlin_ref[...], preferred_element_type=f32)
    bias = (jnp.dot(b1_ref[...][None, :], wlin_ref[...],
                    preferred_element_type=f32) + blin_ref[...][None, :])
    g8_ref[...] = jnp.dot(hp, m1p, preferred_element_type=f32)
    base = jnp.dot(ha, m2, preferred_element_type=f32) + bias
    base_ref[...] = jnp.concatenate(
        [base, 1.0 / ca, jnp.zeros((base.shape[0], 3), f32)], axis=1)


def _tc_mid(feat, cnt, xp, xa, w0ra, b0a, w0rp, b0p, w1l, w1r, b1,
            wlin, blin):
    return pl.pallas_call(
        _mid_body,
        grid=(GRID,),
        in_specs=[
            pl.BlockSpec((2, BM, H), lambda i: (0, i, 0)),
            pl.BlockSpec((2, BM, 8), lambda i: (0, i, 0)),
            pl.BlockSpec((BM, D), lambda i: (i, 0)),
            pl.BlockSpec((BM, D), lambda i: (i, 0)),
            pl.BlockSpec((D, H), lambda i: (0, 0)),
            pl.BlockSpec((H,), lambda i: (0,)),
            pl.BlockSpec((D, H), lambda i: (0, 0)),
            pl.BlockSpec((H,), lambda i: (0,)),
            pl.BlockSpec((H, H), lambda i: (0, 0)),
            pl.BlockSpec((H, H), lambda i: (0, 0)),
            pl.BlockSpec((H,), lambda i: (0,)),
            pl.BlockSpec((H, OUT), lambda i: (0, 0)),
            pl.BlockSpec((OUT,), lambda i: (0,)),
        ],
        out_specs=[
            pl.BlockSpec((BM, 8), lambda i: (i, 0)),
            pl.BlockSpec((BM, 8), lambda i: (i, 0)),
        ],
        out_shape=[
            jax.ShapeDtypeStruct((NA, 8), jnp.float32),
            jax.ShapeDtypeStruct((NA, 8), jnp.float32),
        ],
    )(feat, cnt, xp, xa, w0ra, b0a, w0rp, b0p, w1l, w1r, b1, wlin, blin)


def _sc_agg_l1(g8, srcs, dsts, z8):
    @functools.partial(
        pl.kernel,
        out_type=jax.ShapeDtypeStruct((NC, NR, 8), jnp.float32),
        mesh=_sc_mesh(),
        compiler_params=pltpu.CompilerParams(use_tc_tiling_on_sc=False),
        scratch_types=[
            pltpu.VMEM((CH_D, CHUNK), jnp.int32),
            pltpu.VMEM((CH_D, CHUNK), jnp.int32),
            pltpu.VMEM((CHUNK, 8), jnp.float32),
            pltpu.VMEM((CHUNK, 8), jnp.float32),
            pltpu.VMEM_SHARED((NP, 8), jnp.float32),
            pltpu.VMEM_SHARED((NR, 8), jnp.float32),
            pltpu.SemaphoreType.DMA,
            pltpu.SemaphoreType.DMA,
        ],
    )
    def k(g8_h, srcs_h, dsts_h, z8_h, out_o,
          src_v, dst_v, buf_a, buf_b, gtab, acc8, sem_a, sem_b):
        c = lax.axis_index("c")
        s = lax.axis_index("s")
        w = c * NS + s
        pltpu.sync_copy(
            srcs_h.at[1, w // 2, pl.ds((w % 2) * CH_D, CH_D)], src_v)
        pltpu.sync_copy(
            dsts_h.at[1, w // 2, pl.ds((w % 2) * CH_D, CH_D)], dst_v)
        st0 = s * (NP // NS)
        pltpu.sync_copy(g8_h.at[pl.ds(st0, NP // NS)],
                        gtab.at[pl.ds(st0, NP // NS)])
        r0 = s * RPT
        pltpu.sync_copy(z8_h, acc8.at[pl.ds(r0, RPT)])
        plsc.subcore_barrier()

        def pair(i, carry):
            g = i * 2
            cp_a = pltpu.async_copy(gtab.at[src_v.at[g]], buf_a, sem_a)
            cp_b = pltpu.async_copy(gtab.at[src_v.at[g + 1]], buf_b, sem_b)
            cp_a.wait()
            pltpu.sync_copy(buf_a, acc8.at[dst_v.at[g]], add=True)
            cp_b.wait()
            pltpu.sync_copy(buf_b, acc8.at[dst_v.at[g + 1]], add=True)
            return carry

        lax.fori_loop(0, CH_D // 2, pair, 0)
        plsc.subcore_barrier()
        pltpu.sync_copy(acc8.at[pl.ds(r0, RPT)],
                        out_o.at[c, pl.ds(r0, RPT)])

    return k(g8, srcs, dsts, z8)


def _fin_body(s8_ref, base_ref, out_ref):
    ssum = s8_ref[0] + s8_ref[1]
    out_ref[...] = (ssum[:, 0:OUT] * base_ref[:, OUT:OUT + 1]
                    + base_ref[:, 0:OUT])


def _tc_fin(s8, base):
    return pl.pallas_call(
        _fin_body,
        grid=(GRID,),
        in_specs=[
            pl.BlockSpec((2, BM, 8), lambda i: (0, i, 0)),
            pl.BlockSpec((BM, 8), lambda i: (i, 0)),
        ],
        out_specs=pl.BlockSpec((BM, OUT), lambda i: (i, 0)),
        out_shape=jax.ShapeDtypeStruct((NA, OUT), jnp.float32),
    )(s8, base)


def kernel(x_author, x_paper, edge_index_a2p, edge_index_p2a,
           W0l_a2p, b0_a2p, W0r_a2p, W0l_p2a, b0_p2a, W0r_p2a,
           W1l_a2p, b1_a2p, W1r_a2p, W1l_p2a, b1_p2a, W1r_p2a,
           Wlin, blin):
    del W1l_a2p, b1_a2p, W1r_a2p

    s_a2p, d_a2p = _pack_edges(edge_index_a2p)
    s_p2a, d_p2a = _pack_edges(edge_index_p2a)
    srcs_b = jnp.stack([s_a2p, s_p2a])
    dsts_b = jnp.stack([d_a2p, d_p2a])

    z64 = jnp.zeros((RPT, H), jnp.float32)
    z8 = jnp.zeros((RPT, 8), jnp.float32)
    ones8 = jnp.ones((CHUNK, 8), jnp.float32)

    tabs = _make_tables(x_author, x_paper, W0l_a2p, W0l_p2a)
    feat, cnt = _sc_agg_l0(tabs, srcs_b, dsts_b, z64, z8, ones8)
    g8, base = _tc_mid(feat, cnt, x_paper, x_author, W0r_a2p, b0_a2p,
                       W0r_p2a, b0_p2a, W1l_p2a, W1r_p2a, b1_p2a,
                       Wlin, blin)
    s8 = _sc_agg_l1(g8, srcs_b, dsts_b, z8)
    return _tc_fin(s8, base)


# speedup vs baseline: 1.1677x; 1.1677x over previous
import functools

import jax
import jax.numpy as jnp
from jax import lax
from jax.experimental import pallas as pl
from jax.experimental.pallas import tpu as pltpu
from jax.experimental.pallas import tpu_sc as plsc

NA = 10000
NP = 10000
E = 320000
D = 128
H = 64
OUT = 4

NC = 2
NS = 16
NR = 10240
RPT = NR // NS
CHUNK = 128
CH_B = 160
CH_D = 80
NBUF = 4
BM = 1000
GRID = NA // BM


def _pack_edges(ei, shards, chunks):
    tot = shards * chunks * CHUNK
    padn = tot - E
    src = jnp.concatenate(
        [ei[0].astype(jnp.int32), jnp.zeros((padn,), jnp.int32)])
    dump = NA + 48 + (jnp.arange(padn, dtype=jnp.int32) % 128)
    dst = jnp.concatenate([ei[1].astype(jnp.int32), dump])
    return (src.reshape(shards, chunks, CHUNK),
            dst.reshape(shards, chunks, CHUNK))


def _tab_body(xa_ref, xp_ref, wa_ref, wp_ref, out_ref):
    out_ref[0] = jnp.dot(xa_ref[...], wa_ref[...],
                         preferred_element_type=jnp.float32)
    out_ref[1] = jnp.dot(xp_ref[...], wp_ref[...],
                         preferred_element_type=jnp.float32)


def _make_tables(xa, xp, wa, wp):
    return pl.pallas_call(
        _tab_body,
        grid=(GRID,),
        in_specs=[
            pl.BlockSpec((BM, D), lambda i: (i, 0)),
            pl.BlockSpec((BM, D), lambda i: (i, 0)),
            pl.BlockSpec((D, H), lambda i: (0, 0)),
            pl.BlockSpec((D, H), lambda i: (0, 0)),
        ],
        out_specs=pl.BlockSpec((2, BM, H), lambda i: (0, i, 0)),
        out_shape=jax.ShapeDtypeStruct((2, NA, H), jnp.float32),
    )(xa, xp, wa, wp)


def _sc_mesh():
    return plsc.VectorSubcoreMesh(
        core_axis_name="c", subcore_axis_name="s",
        num_cores=NC, num_subcores=NS)


def _sc_agg_l0(tabs, srcs, dsts, z64, z8, ones8):
    @functools.partial(
        pl.kernel,
        out_type=[jax.ShapeDtypeStruct((NC, NR, H), jnp.float32),
                  jax.ShapeDtypeStruct((NC, NR, 8), jnp.float32)],
        mesh=_sc_mesh(),
        compiler_params=pltpu.CompilerParams(use_tc_tiling_on_sc=False),
        scratch_types=[
            pltpu.VMEM((CH_B, CHUNK), jnp.int32),
            pltpu.VMEM((CH_B, CHUNK), jnp.int32),
            [pltpu.VMEM((CHUNK, H), jnp.float32)] * NBUF,
            pltpu.VMEM((CHUNK, 8), jnp.float32),
            pltpu.VMEM_SHARED((NR, H), jnp.float32),
            pltpu.VMEM_SHARED((NR, 8), jnp.float32),
            [pltpu.SemaphoreType.DMA] * NBUF,
        ],
    )
    def k(tabs_h, srcs_h, dsts_h, z64_h, z8_h, ones8_h, feat_o, cnt_o,
          src_v, dst_v, bufs, ones_v, acc_f, acc_c, sems):
        c = lax.axis_index("c")
        s = lax.axis_index("s")
        pltpu.sync_copy(srcs_h.at[c, s], src_v)
        pltpu.sync_copy(dsts_h.at[c, s], dst_v)
        pltpu.sync_copy(ones8_h, ones_v)
        r0 = s * RPT
        pltpu.sync_copy(z64_h, acc_f.at[pl.ds(r0, RPT)])
        pltpu.sync_copy(z8_h, acc_c.at[pl.ds(r0, RPT)])
        plsc.subcore_barrier()
        tab = tabs_h.at[c]

        for b in range(NBUF):
            pltpu.async_copy(tab.at[src_v.at[b]], bufs[b], sems[b])

        def grp(i, carry):
            g0 = i * NBUF
            for b in range(NBUF):
                g = g0 + b
                pltpu.make_async_copy(
                    tab.at[src_v.at[g]], bufs[b], sems[b]).wait()
                pltpu.sync_copy(bufs[b], acc_f.at[dst_v.at[g]], add=True)
                pltpu.sync_copy(ones_v, acc_c.at[dst_v.at[g]], add=True)

                @pl.when(g + NBUF < CH_B)
                def _():
                    pltpu.async_copy(
                        tab.at[src_v.at[g + NBUF]], bufs[b], sems[b])
            return carry

        lax.fori_loop(0, CH_B // NBUF, grp, 0)
        plsc.subcore_barrier()
        pltpu.sync_copy(acc_f.at[pl.ds(r0, RPT)],
                        feat_o.at[c, pl.ds(r0, RPT)])
        pltpu.sync_copy(acc_c.at[pl.ds(r0, RPT)],
                        cnt_o.at[c, pl.ds(r0, RPT)])

    return k(tabs, srcs, dsts, z64, z8, ones8)


def _mid_body(feat_ref, cnt_ref, xp_ref, xa_ref, w0ra_ref, b0a_ref,
              w0rp_ref, b0p_ref, w1l_ref, w1r_ref, b1_ref, wlin_ref,
              blin_ref, g8_ref, base_ref):
    f32 = jnp.float32
    cp = jnp.maximum(cnt_ref[0][:, 0:1], 1.0)
    ca = jnp.maximum(cnt_ref[1][:, 0:1], 1.0)
    hp = (feat_ref[0] / cp + b0a_ref[...][None, :]
          + jnp.dot(xp_ref[...], w0ra_ref[...], preferred_element_type=f32))
    hp = jnp.where(hp >= 0, hp, 0.01 * hp)
    ha = (feat_ref[1] / ca + b0p_ref[...][None, :]
          + jnp.dot(xa_ref[...], w0rp_ref[...], preferred_element_type=f32))
    ha = jnp.where(ha >= 0, ha, 0.01 * ha)
    m1 = jnp.dot(w1l_ref[...], wlin_ref[...], preferred_element_type=f32)
    m1p = jnp.concatenate([m1, jnp.zeros_like(m1)], axis=1)
    m2 = jnp.dot(w1r_ref[...], wlin_ref[...], preferred_element_type=f32)
    bias = (jnp.dot(b1_ref[...][None, :], wlin_ref[...],
                    preferred_element_type=f32) + blin_ref[...][None, :])
    g8_ref[...] = jnp.dot(hp, m1p, preferred_element_type=f32)
    base_ref[...] = jnp.dot(ha, m2, preferred_element_type=f32) + bias


def _tc_mid(feat, cnt, xp, xa, w0ra, b0a, w0rp, b0p, w1l, w1r, b1,
            wlin, blin):
    return pl.pallas_call(
        _mid_body,
        grid=(GRID,),
        in_specs=[
            pl.BlockSpec((2, BM, H), lambda i: (0, i, 0)),
            pl.BlockSpec((2, BM, 8), lambda i: (0, i, 0)),
            pl.BlockSpec((BM, D), lambda i: (i, 0)),
            pl.BlockSpec((BM, D), lambda i: (i, 0)),
            pl.BlockSpec((D, H), lambda i: (0, 0)),
            pl.BlockSpec((H,), lambda i: (0,)),
            pl.BlockSpec((D, H), lambda i: (0, 0)),
            pl.BlockSpec((H,), lambda i: (0,)),
            pl.BlockSpec((H, H), lambda i: (0, 0)),
            pl.BlockSpec((H, H), lambda i: (0, 0)),
            pl.BlockSpec((H,), lambda i: (0,)),
            pl.BlockSpec((H, OUT), lambda i: (0, 0)),
            pl.BlockSpec((OUT,), lambda i: (0,)),
        ],
        out_specs=[
            pl.BlockSpec((BM, 8), lambda i: (i, 0)),
            pl.BlockSpec((BM, OUT), lambda i: (i, 0)),
        ],
        out_shape=[
            jax.ShapeDtypeStruct((NA, 8), jnp.float32),
            jax.ShapeDtypeStruct((NA, OUT), jnp.float32),
        ],
    )(feat, cnt, xp, xa, w0ra, b0a, w0rp, b0p, w1l, w1r, b1, wlin, blin)


def _sc_agg_l1(g8, srcs, dsts, z8):
    @functools.partial(
        pl.kernel,
        out_type=jax.ShapeDtypeStruct((NC, NR, 8), jnp.float32),
        mesh=_sc_mesh(),
        compiler_params=pltpu.CompilerParams(use_tc_tiling_on_sc=False),
        scratch_types=[
            pltpu.VMEM((CH_D, CHUNK), jnp.int32),
            pltpu.VMEM((CH_D, CHUNK), jnp.int32),
            [pltpu.VMEM((CHUNK, 8), jnp.float32)] * NBUF,
            pltpu.VMEM_SHARED((NP, 8), jnp.float32),
            pltpu.VMEM_SHARED((NR, 8), jnp.float32),
            [pltpu.SemaphoreType.DMA] * NBUF,
        ],
    )
    def k(g8_h, srcs_h, dsts_h, z8_h, out_o,
          src_v, dst_v, bufs, gtab, acc8, sems):
        c = lax.axis_index("c")
        s = lax.axis_index("s")
        w = c * NS + s
        pltpu.sync_copy(srcs_h.at[w], src_v)
        pltpu.sync_copy(dsts_h.at[w], dst_v)
        st0 = s * (NP // NS)
        pltpu.sync_copy(g8_h.at[pl.ds(st0, NP // NS)],
                        gtab.at[pl.ds(st0, NP // NS)])
        r0 = s * RPT
        pltpu.sync_copy(z8_h, acc8.at[pl.ds(r0, RPT)])
        plsc.subcore_barrier()

        for b in range(NBUF):
            pltpu.async_copy(gtab.at[src_v.at[b]], bufs[b], sems[b])

        def grp(i, carry):
            g0 = i * NBUF
            for b in range(NBUF):
                g = g0 + b
                pltpu.make_async_copy(
                    gtab.at[src_v.at[g]], bufs[b], sems[b]).wait()
                pltpu.sync_copy(bufs[b], acc8.at[dst_v.at[g]], add=True)

                @pl.when(g + NBUF < CH_D)
                def _():
                    pltpu.async_copy(
                        gtab.at[src_v.at[g + NBUF]], bufs[b], sems[b])
            return carry

        lax.fori_loop(0, CH_D // NBUF, grp, 0)
        plsc.subcore_barrier()
        pltpu.sync_copy(acc8.at[pl.ds(r0, RPT)],
                        out_o.at[c, pl.ds(r0, RPT)])

    return k(g8, srcs, dsts, z8)


def _fin_body(s8_ref, cnt_ref, base_ref, out_ref):
    ssum = s8_ref[0] + s8_ref[1]
    ca = jnp.maximum(cnt_ref[1][:, 0:1], 1.0)
    out_ref[...] = ssum[:, 0:OUT] / ca + base_ref[...]


def _tc_fin(s8, cnt, base):
    return pl.pallas_call(
        _fin_body,
        grid=(GRID,),
        in_specs=[
            pl.BlockSpec((2, BM, 8), lambda i: (0, i, 0)),
            pl.BlockSpec((2, BM, 8), lambda i: (0, i, 0)),
            pl.BlockSpec((BM, OUT), lambda i: (i, 0)),
        ],
        out_specs=pl.BlockSpec((BM, OUT), lambda i: (i, 0)),
        out_shape=jax.ShapeDtypeStruct((NA, OUT), jnp.float32),
    )(s8, cnt, base)


def kernel(x_author, x_paper, edge_index_a2p, edge_index_p2a,
           W0l_a2p, b0_a2p, W0r_a2p, W0l_p2a, b0_p2a, W0r_p2a,
           W1l_a2p, b1_a2p, W1r_a2p, W1l_p2a, b1_p2a, W1r_p2a,
           Wlin, blin):
    del W1l_a2p, b1_a2p, W1r_a2p

    s_a2p, d_a2p = _pack_edges(edge_index_a2p, NS, CH_B)
    s_p2a, d_p2a = _pack_edges(edge_index_p2a, NS, CH_B)
    srcs_b = jnp.stack([s_a2p, s_p2a])
    dsts_b = jnp.stack([d_a2p, d_p2a])
    s_d, d_d = _pack_edges(edge_index_p2a, NC * NS, CH_D)

    z64 = jnp.zeros((RPT, H), jnp.float32)
    z8 = jnp.zeros((RPT, 8), jnp.float32)
    ones8 = jnp.ones((CHUNK, 8), jnp.float32)

    tabs = _make_tables(x_author, x_paper, W0l_a2p, W0l_p2a)
    feat, cnt = _sc_agg_l0(tabs, srcs_b, dsts_b, z64, z8, ones8)
    g8, base = _tc_mid(feat, cnt, x_paper, x_author, W0r_a2p, b0_a2p,
                       W0r_p2a, b0_p2a, W1l_p2a, W1r_p2a, b1_p2a,
                       Wlin, blin)
    s8 = _sc_agg_l1(g8, s_d, d_d, z8)
    return _tc_fin(s8, cnt, base)


# speedup vs baseline: 1.3062x; 1.1186x over previous
import functools

import jax
import jax.numpy as jnp
from jax import lax
from jax.experimental import pallas as pl
from jax.experimental.pallas import tpu as pltpu
from jax.experimental.pallas import tpu_sc as plsc

NA = 10000
NP = 10000
E = 320000
D = 128
H = 64
OUT = 4

NC = 2
NS = 16
NR = 10240
RPT = NR // NS
CHUNK = 128
CH_B = 158
CH_D = 80
NBUF = 4
BM = 1000
GRID = NA // BM


def _pack_edges(ei, shards, chunks):
    tot = shards * chunks * CHUNK
    padn = tot - E
    src = jnp.concatenate(
        [ei[0].astype(jnp.int32), jnp.zeros((padn,), jnp.int32)])
    dump = NA + 48 + (jnp.arange(padn, dtype=jnp.int32) % 128)
    dst = jnp.concatenate([ei[1].astype(jnp.int32), dump])
    return (src.reshape(shards, chunks, CHUNK),
            dst.reshape(shards, chunks, CHUNK))


def _tab_body(xa_ref, xp_ref, wa_ref, wp_ref, out_ref):
    out_ref[0] = jnp.dot(xa_ref[...], wa_ref[...],
                         preferred_element_type=jnp.float32)
    out_ref[1] = jnp.dot(xp_ref[...], wp_ref[...],
                         preferred_element_type=jnp.float32)


def _make_tables(xa, xp, wa, wp):
    return pl.pallas_call(
        _tab_body,
        grid=(GRID,),
        in_specs=[
            pl.BlockSpec((BM, D), lambda i: (i, 0)),
            pl.BlockSpec((BM, D), lambda i: (i, 0)),
            pl.BlockSpec((D, H), lambda i: (0, 0)),
            pl.BlockSpec((D, H), lambda i: (0, 0)),
        ],
        out_specs=pl.BlockSpec((2, BM, H), lambda i: (0, i, 0)),
        out_shape=jax.ShapeDtypeStruct((2, NA, H), jnp.float32),
    )(xa, xp, wa, wp)


def _sc_mesh():
    return plsc.VectorSubcoreMesh(
        core_axis_name="c", subcore_axis_name="s",
        num_cores=NC, num_subcores=NS)


def _sc_agg_l0(tabs, srcs, dsts, z64, z8, ones8):
    @functools.partial(
        pl.kernel,
        out_type=[jax.ShapeDtypeStruct((NC, NR, H), jnp.float32),
                  jax.ShapeDtypeStruct((NC, NR, 8), jnp.float32)],
        mesh=_sc_mesh(),
        compiler_params=pltpu.CompilerParams(use_tc_tiling_on_sc=False),
        scratch_types=[
            pltpu.VMEM((CH_B, CHUNK), jnp.int32),
            pltpu.VMEM((CH_B, CHUNK), jnp.int32),
            pltpu.VMEM((CHUNK, H), jnp.float32),
            pltpu.VMEM((CHUNK, H), jnp.float32),
            pltpu.VMEM((CHUNK, 8), jnp.float32),
            pltpu.VMEM_SHARED((NR, H), jnp.float32),
            pltpu.VMEM_SHARED((NR, 8), jnp.float32),
            pltpu.SemaphoreType.DMA,
            pltpu.SemaphoreType.DMA,
        ],
    )
    def k(tabs_h, srcs_h, dsts_h, z64_h, z8_h, ones8_h, feat_o, cnt_o,
          src_v, dst_v, buf_a, buf_b, ones_v, acc_f, acc_c, sem_a, sem_b):
        c = lax.axis_index("c")
        s = lax.axis_index("s")
        pltpu.sync_copy(srcs_h.at[c, s], src_v)
        pltpu.sync_copy(dsts_h.at[c, s], dst_v)
        pltpu.sync_copy(ones8_h, ones_v)
        r0 = s * RPT
        pltpu.sync_copy(z64_h, acc_f.at[pl.ds(r0, RPT)])
        pltpu.sync_copy(z8_h, acc_c.at[pl.ds(r0, RPT)])
        plsc.subcore_barrier()
        tab = tabs_h.at[c]

        def pair(i, carry):
            g = i * 2
            cp_a = pltpu.async_copy(tab.at[src_v.at[g]], buf_a, sem_a)
            cp_b = pltpu.async_copy(tab.at[src_v.at[g + 1]], buf_b, sem_b)
            cp_a.wait()
            pltpu.sync_copy(buf_a, acc_f.at[dst_v.at[g]], add=True)
            pltpu.sync_copy(ones_v, acc_c.at[dst_v.at[g]], add=True)
            cp_b.wait()
            pltpu.sync_copy(buf_b, acc_f.at[dst_v.at[g + 1]], add=True)
            pltpu.sync_copy(ones_v, acc_c.at[dst_v.at[g + 1]], add=True)
            return carry

        lax.fori_loop(0, CH_B // 2, pair, 0)
        plsc.subcore_barrier()
        pltpu.sync_copy(acc_f.at[pl.ds(r0, RPT)],
                        feat_o.at[c, pl.ds(r0, RPT)])
        pltpu.sync_copy(acc_c.at[pl.ds(r0, RPT)],
                        cnt_o.at[c, pl.ds(r0, RPT)])

    return k(tabs, srcs, dsts, z64, z8, ones8)


def _mid_body(feat_ref, cnt_ref, xp_ref, xa_ref, w0ra_ref, b0a_ref,
              w0rp_ref, b0p_ref, w1l_ref, w1r_ref, b1_ref, wlin_ref,
              blin_ref, g8_ref, base_ref):
    f32 = jnp.float32
    cp = jnp.maximum(cnt_ref[0][:, 0:1], 1.0)
    ca = jnp.maximum(cnt_ref[1][:, 0:1], 1.0)
    hp = (feat_ref[0] / cp + b0a_ref[...][None, :]
          + jnp.dot(xp_ref[...], w0ra_ref[...], preferred_element_type=f32))
    hp = jnp.where(hp >= 0, hp, 0.01 * hp)
    ha = (feat_ref[1] / ca + b0p_ref[...][None, :]
          + jnp.dot(xa_ref[...], w0rp_ref[...], preferred_element_type=f32))
    ha = jnp.where(ha >= 0, ha, 0.01 * ha)
    m1 = jnp.dot(w1l_ref[...], wlin_ref[...], preferred_element_type=f32)
    m1p = jnp.concatenate([m1, jnp.zeros_like(m1)], axis=1)
    m2 = jnp.dot(w1r_ref[...], wlin_ref[...], preferred_element_type=f32)
    bias = (jnp.dot(b1_ref[...][None, :], wlin_ref[...],
                    preferred_element_type=f32) + blin_ref[...][None, :])
    g8_ref[...] = jnp.dot(hp, m1p, preferred_element_type=f32)
    base = jnp.dot(ha, m2, preferred_element_type=f32) + bias
    base_ref[...] = jnp.concatenate(
        [base, 1.0 / ca, jnp.zeros((base.shape[0], 3), f32)], axis=1)


def _tc_mid(feat, cnt, xp, xa, w0ra, b0a, w0rp, b0p, w1l, w1r, b1,
            wlin, blin):
    return pl.pallas_call(
        _mid_body,
        grid=(GRID,),
        in_specs=[
            pl.BlockSpec((2, BM, H), lambda i: (0, i, 0)),
            pl.BlockSpec((2, BM, 8), lambda i: (0, i, 0)),
            pl.BlockSpec((BM, D), lambda i: (i, 0)),
            pl.BlockSpec((BM, D), lambda i: (i, 0)),
            pl.BlockSpec((D, H), lambda i: (0, 0)),
            pl.BlockSpec((H,), lambda i: (0,)),
            pl.BlockSpec((D, H), lambda i: (0, 0)),
            pl.BlockSpec((H,), lambda i: (0,)),
            pl.BlockSpec((H, H), lambda i: (0, 0)),
            pl.BlockSpec((H, H), lambda i: (0, 0)),
            pl.BlockSpec((H,), lambda i: (0,)),
            pl.BlockSpec((H, OUT), lambda i: (0, 0)),
            pl.BlockSpec((OUT,), lambda i: (0,)),
        ],
        out_specs=[
            pl.BlockSpec((BM, 8), lambda i: (i, 0)),
            pl.BlockSpec((BM, 8), lambda i: (i, 0)),
        ],
        out_shape=[
            jax.ShapeDtypeStruct((NA, 8), jnp.float32),
            jax.ShapeDtypeStruct((NA, 8), jnp.float32),
        ],
    )(feat, cnt, xp, xa, w0ra, b0a, w0rp, b0p, w1l, w1r, b1, wlin, blin)


def _sc_agg_l1(g8, srcs, dsts, z8):
    @functools.partial(
        pl.kernel,
        out_type=jax.ShapeDtypeStruct((NC, NR, 8), jnp.float32),
        mesh=_sc_mesh(),
        compiler_params=pltpu.CompilerParams(use_tc_tiling_on_sc=False),
        scratch_types=[
            pltpu.VMEM((CH_D, CHUNK), jnp.int32),
            pltpu.VMEM((CH_D, CHUNK), jnp.int32),
            [pltpu.VMEM((CHUNK, 8), jnp.float32)] * NBUF,
            pltpu.VMEM_SHARED((NP, 8), jnp.float32),
            pltpu.VMEM_SHARED((NR, 8), jnp.float32),
            [pltpu.SemaphoreType.DMA] * NBUF,
        ],
    )
    def k(g8_h, srcs_h, dsts_h, z8_h, out_o,
          src_v, dst_v, bufs, gtab, acc8, sems):
        c = lax.axis_index("c")
        s = lax.axis_index("s")
        w = c * NS + s
        pltpu.sync_copy(srcs_h.at[w], src_v)
        pltpu.sync_copy(dsts_h.at[w], dst_v)
        st0 = s * (NP // NS)
        pltpu.sync_copy(g8_h.at[pl.ds(st0, NP // NS)],
                        gtab.at[pl.ds(st0, NP // NS)])
        r0 = s * RPT
        pltpu.sync_copy(z8_h, acc8.at[pl.ds(r0, RPT)])
        plsc.subcore_barrier()

        for b in range(NBUF):
            pltpu.async_copy(gtab.at[src_v.at[b]], bufs[b], sems[b])

        def grp(i, carry):
            g0 = i * NBUF
            for b in range(NBUF):
                g = g0 + b
                pltpu.make_async_copy(
                    gtab.at[src_v.at[g]], bufs[b], sems[b]).wait()
                pltpu.sync_copy(bufs[b], acc8.at[dst_v.at[g]], add=True)

                @pl.when(g + NBUF < CH_D)
                def _():
                    pltpu.async_copy(
                        gtab.at[src_v.at[g + NBUF]], bufs[b], sems[b])
            return carry

        lax.fori_loop(0, CH_D // NBUF, grp, 0)
        plsc.subcore_barrier()
        pltpu.sync_copy(acc8.at[pl.ds(r0, RPT)],
                        out_o.at[c, pl.ds(r0, RPT)])

    return k(g8, srcs, dsts, z8)


def _fin_body(s8_ref, base_ref, out_ref):
    ssum = s8_ref[0] + s8_ref[1]
    out_ref[...] = (ssum[:, 0:OUT] * base_ref[:, OUT:OUT + 1]
                    + base_ref[:, 0:OUT])


def _tc_fin(s8, base):
    return pl.pallas_call(
        _fin_body,
        grid=(GRID,),
        in_specs=[
            pl.BlockSpec((2, BM, 8), lambda i: (0, i, 0)),
            pl.BlockSpec((BM, 8), lambda i: (i, 0)),
        ],
        out_specs=pl.BlockSpec((BM, OUT), lambda i: (i, 0)),
        out_shape=jax.ShapeDtypeStruct((NA, OUT), jnp.float32),
    )(s8, base)


def kernel(x_author, x_paper, edge_index_a2p, edge_index_p2a,
           W0l_a2p, b0_a2p, W0r_a2p, W0l_p2a, b0_p2a, W0r_p2a,
           W1l_a2p, b1_a2p, W1r_a2p, W1l_p2a, b1_p2a, W1r_p2a,
           Wlin, blin):
    del W1l_a2p, b1_a2p, W1r_a2p

    s_a2p, d_a2p = _pack_edges(edge_index_a2p, NS, CH_B)
    s_p2a, d_p2a = _pack_edges(edge_index_p2a, NS, CH_B)
    srcs_b = jnp.stack([s_a2p, s_p2a])
    dsts_b = jnp.stack([d_a2p, d_p2a])
    s_d, d_d = _pack_edges(edge_index_p2a, NC * NS, CH_D)

    z64 = jnp.zeros((RPT, H), jnp.float32)
    z8 = jnp.zeros((RPT, 8), jnp.float32)
    ones8 = jnp.ones((CHUNK, 8), jnp.float32)

    tabs = _make_tables(x_author, x_paper, W0l_a2p, W0l_p2a)
    feat, cnt = _sc_agg_l0(tabs, srcs_b, dsts_b, z64, z8, ones8)
    g8, base = _tc_mid(feat, cnt, x_paper, x_author, W0r_a2p, b0_a2p,
                       W0r_p2a, b0_p2a, W1l_p2a, W1r_p2a, b1_p2a,
                       Wlin, blin)
    s8 = _sc_agg_l1(g8, s_d, d_d, z8)
    return _tc_fin(s8, base)


# speedup vs baseline: 1.3266x; 1.0156x over previous
import functools

import jax
import jax.numpy as jnp
from jax import lax
from jax.experimental import pallas as pl
from jax.experimental.pallas import tpu as pltpu
from jax.experimental.pallas import tpu_sc as plsc

NA = 10000
NP = 10000
E = 320000
D = 128
H = 64
OUT = 4

NC = 2
NS = 16
NR = 10240
RPT = NR // NS
CHUNK = 128
CH_B = 158
CH_D = 80
NBUF = 4
BM = 1000
GRID = NA // BM


def _pack_edges(ei, shards, chunks):
    tot = shards * chunks * CHUNK
    padn = tot - E
    src = jnp.concatenate(
        [ei[0].astype(jnp.int32), jnp.zeros((padn,), jnp.int32)])
    dump = NA + 48 + (jnp.arange(padn, dtype=jnp.int32) % 128)
    dst = jnp.concatenate([ei[1].astype(jnp.int32), dump])
    return (src.reshape(shards, chunks, CHUNK),
            dst.reshape(shards, chunks, CHUNK))


def _pack_edge_pair(ei_a, ei_b):
    tot = NS * CH_B * CHUNK
    padn = tot - E
    zpad = jnp.zeros((padn,), jnp.int32)
    dump = NA + 48 + (jnp.arange(padn, dtype=jnp.int32) % 128)
    src = jnp.concatenate([ei_a[0].astype(jnp.int32), zpad,
                           ei_b[0].astype(jnp.int32), zpad])
    dst = jnp.concatenate([ei_a[1].astype(jnp.int32), dump,
                           ei_b[1].astype(jnp.int32), dump])
    return (src.reshape(2, NS, CH_B, CHUNK),
            dst.reshape(2, NS, CH_B, CHUNK))


def _tab_body(xa_ref, xp_ref, wa_ref, wp_ref, out_ref):
    out_ref[0] = jnp.dot(xa_ref[...], wa_ref[...],
                         preferred_element_type=jnp.float32)
    out_ref[1] = jnp.dot(xp_ref[...], wp_ref[...],
                         preferred_element_type=jnp.float32)


def _make_tables(xa, xp, wa, wp):
    return pl.pallas_call(
        _tab_body,
        grid=(GRID,),
        in_specs=[
            pl.BlockSpec((BM, D), lambda i: (i, 0)),
            pl.BlockSpec((BM, D), lambda i: (i, 0)),
            pl.BlockSpec((D, H), lambda i: (0, 0)),
            pl.BlockSpec((D, H), lambda i: (0, 0)),
        ],
        out_specs=pl.BlockSpec((2, BM, H), lambda i: (0, i, 0)),
        out_shape=jax.ShapeDtypeStruct((2, NA, H), jnp.float32),
    )(xa, xp, wa, wp)


def _sc_mesh():
    return plsc.VectorSubcoreMesh(
        core_axis_name="c", subcore_axis_name="s",
        num_cores=NC, num_subcores=NS)


def _sc_agg_l0(tabs, srcs, dsts, z64, z8, ones8):
    @functools.partial(
        pl.kernel,
        out_type=[jax.ShapeDtypeStruct((NC, NR, H), jnp.float32),
                  jax.ShapeDtypeStruct((NC, NR, 8), jnp.float32)],
        mesh=_sc_mesh(),
        compiler_params=pltpu.CompilerParams(use_tc_tiling_on_sc=False),
        scratch_types=[
            pltpu.VMEM((CH_B, CHUNK), jnp.int32),
            pltpu.VMEM((CH_B, CHUNK), jnp.int32),
            pltpu.VMEM((CHUNK, H), jnp.float32),
            pltpu.VMEM((CHUNK, H), jnp.float32),
            pltpu.VMEM((CHUNK, 8), jnp.float32),
            pltpu.VMEM_SHARED((NR, H), jnp.float32),
            pltpu.VMEM_SHARED((NR, 8), jnp.float32),
            pltpu.SemaphoreType.DMA,
            pltpu.SemaphoreType.DMA,
        ],
    )
    def k(tabs_h, srcs_h, dsts_h, z64_h, z8_h, ones8_h, feat_o, cnt_o,
          src_v, dst_v, buf_a, buf_b, ones_v, acc_f, acc_c, sem_a, sem_b):
        c = lax.axis_index("c")
        s = lax.axis_index("s")
        pltpu.sync_copy(srcs_h.at[c, s], src_v)
        pltpu.sync_copy(dsts_h.at[c, s], dst_v)
        pltpu.sync_copy(ones8_h, ones_v)
        r0 = s * RPT
        pltpu.sync_copy(z64_h, acc_f.at[pl.ds(r0, RPT)])
        pltpu.sync_copy(z8_h, acc_c.at[pl.ds(r0, RPT)])
        plsc.subcore_barrier()
        tab = tabs_h.at[c]

        def pair(i, carry):
            g = i * 2
            cp_a = pltpu.async_copy(tab.at[src_v.at[g]], buf_a, sem_a)
            cp_b = pltpu.async_copy(tab.at[src_v.at[g + 1]], buf_b, sem_b)
            cp_a.wait()
            pltpu.sync_copy(buf_a, acc_f.at[dst_v.at[g]], add=True)
            pltpu.sync_copy(ones_v, acc_c.at[dst_v.at[g]], add=True)
            cp_b.wait()
            pltpu.sync_copy(buf_b, acc_f.at[dst_v.at[g + 1]], add=True)
            pltpu.sync_copy(ones_v, acc_c.at[dst_v.at[g + 1]], add=True)
            return carry

        lax.fori_loop(0, CH_B // 2, pair, 0)
        plsc.subcore_barrier()
        pltpu.sync_copy(acc_f.at[pl.ds(r0, RPT)],
                        feat_o.at[c, pl.ds(r0, RPT)])
        pltpu.sync_copy(acc_c.at[pl.ds(r0, RPT)],
                        cnt_o.at[c, pl.ds(r0, RPT)])

    return k(tabs, srcs, dsts, z64, z8, ones8)


def _mid_body(feat_ref, cnt_ref, xp_ref, xa_ref, w0ra_ref, b0a_ref,
              w0rp_ref, b0p_ref, w1l_ref, w1r_ref, b1_ref, wlin_ref,
              blin_ref, g8_ref, base_ref):
    f32 = jnp.float32
    cp = jnp.maximum(cnt_ref[0][:, 0:1], 1.0)
    ca = jnp.maximum(cnt_ref[1][:, 0:1], 1.0)
    hp = (feat_ref[0] / cp + b0a_ref[...][None, :]
          + jnp.dot(xp_ref[...], w0ra_ref[...], preferred_element_type=f32))
    hp = jnp.where(hp >= 0, hp, 0.01 * hp)
    ha = (feat_ref[1] / ca + b0p_ref[...][None, :]
          + jnp.dot(xa_ref[...], w0rp_ref[...], preferred_element_type=f32))
    ha = jnp.where(ha >= 0, ha, 0.01 * ha)
    m1 = jnp.dot(w1l_ref[...], wlin_ref[...], preferred_element_type=f32)
    m1p = jnp.concatenate([m1, jnp.zeros_like(m1)], axis=1)
    m2 = jnp.dot(w1r_ref[...], wlin_ref[...], preferred_element_type=f32)
    bias = (jnp.dot(b1_ref[...][None, :], wlin_ref[...],
                    preferred_element_type=f32) + blin_ref[...][None, :])
    g8_ref[...] = jnp.dot(hp, m1p, preferred_element_type=f32)
    base = jnp.dot(ha, m2, preferred_element_type=f32) + bias
    base_ref[...] = jnp.concatenate(
        [base, 1.0 / ca, jnp.zeros((base.shape[0], 3), f32)], axis=1)


def _tc_mid(feat, cnt, xp, xa, w0ra, b0a, w0rp, b0p, w1l, w1r, b1,
            wlin, blin):
    return pl.pallas_call(
        _mid_body,
        grid=(GRID,),
        in_specs=[
            pl.BlockSpec((2, BM, H), lambda i: (0, i, 0)),
            pl.BlockSpec((2, BM, 8), lambda i: (0, i, 0)),
            pl.BlockSpec((BM, D), lambda i: (i, 0)),
            pl.BlockSpec((BM, D), lambda i: (i, 0)),
            pl.BlockSpec((D, H), lambda i: (0, 0)),
            pl.BlockSpec((H,), lambda i: (0,)),
            pl.BlockSpec((D, H), lambda i: (0, 0)),
            pl.BlockSpec((H,), lambda i: (0,)),
            pl.BlockSpec((H, H), lambda i: (0, 0)),
            pl.BlockSpec((H, H), lambda i: (0, 0)),
            pl.BlockSpec((H,), lambda i: (0,)),
            pl.BlockSpec((H, OUT), lambda i: (0, 0)),
            pl.BlockSpec((OUT,), lambda i: (0,)),
        ],
        out_specs=[
            pl.BlockSpec((BM, 8), lambda i: (i, 0)),
            pl.BlockSpec((BM, 8), lambda i: (i, 0)),
        ],
        out_shape=[
            jax.ShapeDtypeStruct((NA, 8), jnp.float32),
            jax.ShapeDtypeStruct((NA, 8), jnp.float32),
        ],
    )(feat, cnt, xp, xa, w0ra, b0a, w0rp, b0p, w1l, w1r, b1, wlin, blin)


def _sc_agg_l1(g8, srcs, dsts, z8):
    @functools.partial(
        pl.kernel,
        out_type=jax.ShapeDtypeStruct((NC, NR, 8), jnp.float32),
        mesh=_sc_mesh(),
        compiler_params=pltpu.CompilerParams(use_tc_tiling_on_sc=False),
        scratch_types=[
            pltpu.VMEM((CH_D, CHUNK), jnp.int32),
            pltpu.VMEM((CH_D, CHUNK), jnp.int32),
            [pltpu.VMEM((CHUNK, 8), jnp.float32)] * NBUF,
            pltpu.VMEM_SHARED((NP, 8), jnp.float32),
            pltpu.VMEM_SHARED((NR, 8), jnp.float32),
            [pltpu.SemaphoreType.DMA] * NBUF,
        ],
    )
    def k(g8_h, srcs_h, dsts_h, z8_h, out_o,
          src_v, dst_v, bufs, gtab, acc8, sems):
        c = lax.axis_index("c")
        s = lax.axis_index("s")
        w = c * NS + s
        pltpu.sync_copy(srcs_h.at[w], src_v)
        pltpu.sync_copy(dsts_h.at[w], dst_v)
        st0 = s * (NP // NS)
        pltpu.sync_copy(g8_h.at[pl.ds(st0, NP // NS)],
                        gtab.at[pl.ds(st0, NP // NS)])
        r0 = s * RPT
        pltpu.sync_copy(z8_h, acc8.at[pl.ds(r0, RPT)])
        plsc.subcore_barrier()

        for b in range(NBUF):
            pltpu.async_copy(gtab.at[src_v.at[b]], bufs[b], sems[b])

        def grp(i, carry):
            g0 = i * NBUF
            for b in range(NBUF):
                g = g0 + b
                pltpu.make_async_copy(
                    gtab.at[src_v.at[g]], bufs[b], sems[b]).wait()
                pltpu.sync_copy(bufs[b], acc8.at[dst_v.at[g]], add=True)

                @pl.when(g + NBUF < CH_D)
                def _():
                    pltpu.async_copy(
                        gtab.at[src_v.at[g + NBUF]], bufs[b], sems[b])
            return carry

        lax.fori_loop(0, CH_D // NBUF, grp, 0)
        plsc.subcore_barrier()
        pltpu.sync_copy(acc8.at[pl.ds(r0, RPT)],
                        out_o.at[c, pl.ds(r0, RPT)])

    return k(g8, srcs, dsts, z8)


def _fin_body(s8_ref, base_ref, out_ref):
    ssum = s8_ref[0] + s8_ref[1]
    out_ref[...] = (ssum[:, 0:OUT] * base_ref[:, OUT:OUT + 1]
                    + base_ref[:, 0:OUT])


def _tc_fin(s8, base):
    return pl.pallas_call(
        _fin_body,
        grid=(GRID,),
        in_specs=[
            pl.BlockSpec((2, BM, 8), lambda i: (0, i, 0)),
            pl.BlockSpec((BM, 8), lambda i: (i, 0)),
        ],
        out_specs=pl.BlockSpec((BM, OUT), lambda i: (i, 0)),
        out_shape=jax.ShapeDtypeStruct((NA, OUT), jnp.float32),
    )(s8, base)


def kernel(x_author, x_paper, edge_index_a2p, edge_index_p2a,
           W0l_a2p, b0_a2p, W0r_a2p, W0l_p2a, b0_p2a, W0r_p2a,
           W1l_a2p, b1_a2p, W1r_a2p, W1l_p2a, b1_p2a, W1r_p2a,
           Wlin, blin):
    del W1l_a2p, b1_a2p, W1r_a2p

    srcs_b, dsts_b = _pack_edge_pair(edge_index_a2p, edge_index_p2a)
    s_d, d_d = _pack_edges(edge_index_p2a, NC * NS, CH_D)

    z64 = jnp.zeros((RPT, H), jnp.float32)
    z8 = jnp.zeros((RPT, 8), jnp.float32)
    ones8 = jnp.ones((CHUNK, 8), jnp.float32)

    tabs = _make_tables(x_author, x_paper, W0l_a2p, W0l_p2a)
    feat, cnt = _sc_agg_l0(tabs, srcs_b, dsts_b, z64, z8, ones8)
    g8, base = _tc_mid(feat, cnt, x_paper, x_author, W0r_a2p, b0_a2p,
                       W0r_p2a, b0_p2a, W1l_p2a, W1r_p2a, b1_p2a,
                       Wlin, blin)
    s8 = _sc_agg_l1(g8, s_d, d_d, z8)
    return _tc_fin(s8, base)


# speedup vs baseline: 1.3298x; 1.0024x over previous
import functools

import jax
import jax.numpy as jnp
from jax import lax
from jax.experimental import pallas as pl
from jax.experimental.pallas import tpu as pltpu
from jax.experimental.pallas import tpu_sc as plsc

NA = 10000
NP = 10000
E = 320000
D = 128
H = 64
OUT = 4

NC = 2
NS = 16
NR = 10240
RPT = NR // NS
CHUNK = 128
CH_B = 158
EROWS = E // CHUNK
TAIL = EROWS - (NS - 1) * CH_B
DUMP = NA + 48
CH_D = 80
NBUF = 4
BM = 1000
GRID = NA // BM


def _pack_edges(ei, shards, chunks):
    tot = shards * chunks * CHUNK
    padn = tot - E
    src = jnp.concatenate(
        [ei[0].astype(jnp.int32), jnp.zeros((padn,), jnp.int32)])
    dump = NA + 48 + (jnp.arange(padn, dtype=jnp.int32) % 128)
    dst = jnp.concatenate([ei[1].astype(jnp.int32), dump])
    return (src.reshape(shards, chunks, CHUNK),
            dst.reshape(shards, chunks, CHUNK))


def _pack_edge_pair(ei_a, ei_b):
    tot = NS * CH_B * CHUNK
    padn = tot - E
    zpad = jnp.zeros((padn,), jnp.int32)
    dump = NA + 48 + (jnp.arange(padn, dtype=jnp.int32) % 128)
    src = jnp.concatenate([ei_a[0].astype(jnp.int32), zpad,
                           ei_b[0].astype(jnp.int32), zpad])
    dst = jnp.concatenate([ei_a[1].astype(jnp.int32), dump,
                           ei_b[1].astype(jnp.int32), dump])
    return (src.reshape(2, NS, CH_B, CHUNK),
            dst.reshape(2, NS, CH_B, CHUNK))


def _tab_body(xa_ref, xp_ref, wa_ref, wp_ref, out_ref):
    out_ref[0] = jnp.dot(xa_ref[...], wa_ref[...],
                         preferred_element_type=jnp.float32)
    out_ref[1] = jnp.dot(xp_ref[...], wp_ref[...],
                         preferred_element_type=jnp.float32)


def _make_tables(xa, xp, wa, wp):
    return pl.pallas_call(
        _tab_body,
        grid=(GRID,),
        in_specs=[
            pl.BlockSpec((BM, D), lambda i: (i, 0)),
            pl.BlockSpec((BM, D), lambda i: (i, 0)),
            pl.BlockSpec((D, H), lambda i: (0, 0)),
            pl.BlockSpec((D, H), lambda i: (0, 0)),
        ],
        out_specs=pl.BlockSpec((2, BM, H), lambda i: (0, i, 0)),
        out_shape=jax.ShapeDtypeStruct((2, NA, H), jnp.float32),
    )(xa, xp, wa, wp)


def _sc_mesh():
    return plsc.VectorSubcoreMesh(
        core_axis_name="c", subcore_axis_name="s",
        num_cores=NC, num_subcores=NS)


def _sc_agg_l0(tabs, ea, ep, z64, z8, ones8):
    @functools.partial(
        pl.kernel,
        out_type=[jax.ShapeDtypeStruct((NC, NR, H), jnp.float32),
                  jax.ShapeDtypeStruct((NC, NR, 8), jnp.float32)],
        mesh=_sc_mesh(),
        compiler_params=pltpu.CompilerParams(use_tc_tiling_on_sc=False),
        scratch_types=[
            pltpu.VMEM((CH_B, CHUNK), jnp.int32),
            pltpu.VMEM((CH_B, CHUNK), jnp.int32),
            pltpu.VMEM((CHUNK, H), jnp.float32),
            pltpu.VMEM((CHUNK, H), jnp.float32),
            pltpu.VMEM((CHUNK, 8), jnp.float32),
            pltpu.VMEM_SHARED((NR, H), jnp.float32),
            pltpu.VMEM_SHARED((NR, 8), jnp.float32),
            pltpu.SemaphoreType.DMA,
            pltpu.SemaphoreType.DMA,
        ],
    )
    def k(tabs_h, ea_h, ep_h, z64_h, z8_h, ones8_h, feat_o, cnt_o,
          src_v, dst_v, buf_a, buf_b, ones_v, acc_f, acc_c, sem_a, sem_b):
        c = lax.axis_index("c")
        s = lax.axis_index("s")

        def stage(e_h):
            @pl.when(s < NS - 1)
            def _():
                pltpu.sync_copy(e_h.at[0, pl.ds(s * CH_B, CH_B)], src_v)
                pltpu.sync_copy(e_h.at[1, pl.ds(s * CH_B, CH_B)], dst_v)

            @pl.when(s == NS - 1)
            def _():
                r = (NS - 1) * CH_B
                pltpu.sync_copy(e_h.at[0, pl.ds(r, TAIL)],
                                src_v.at[pl.ds(0, TAIL)])
                pltpu.sync_copy(e_h.at[1, pl.ds(r, TAIL)],
                                dst_v.at[pl.ds(0, TAIL)])
                lane = lax.iota(jnp.int32, 16)
                zero16 = jnp.zeros((16,), jnp.int32)
                for pr in range(TAIL, CH_B):
                    for j in range(CHUNK // 16):
                        src_v[pr, pl.ds(j * 16, 16)] = zero16
                        dst_v[pr, pl.ds(j * 16, 16)] = (
                            DUMP + (j % 8) * 16 + lane)

        @pl.when(c == 0)
        def _():
            stage(ea_h)

        @pl.when(c == 1)
        def _():
            stage(ep_h)

        pltpu.sync_copy(ones8_h, ones_v)
        r0 = s * RPT
        pltpu.sync_copy(z64_h, acc_f.at[pl.ds(r0, RPT)])
        pltpu.sync_copy(z8_h, acc_c.at[pl.ds(r0, RPT)])
        plsc.subcore_barrier()
        tab = tabs_h.at[c]

        def pair(i, carry):
            g = i * 2
            cp_a = pltpu.async_copy(tab.at[src_v.at[g]], buf_a, sem_a)
            cp_b = pltpu.async_copy(tab.at[src_v.at[g + 1]], buf_b, sem_b)
            cp_a.wait()
            pltpu.sync_copy(buf_a, acc_f.at[dst_v.at[g]], add=True)
            pltpu.sync_copy(ones_v, acc_c.at[dst_v.at[g]], add=True)
            cp_b.wait()
            pltpu.sync_copy(buf_b, acc_f.at[dst_v.at[g + 1]], add=True)
            pltpu.sync_copy(ones_v, acc_c.at[dst_v.at[g + 1]], add=True)
            return carry

        lax.fori_loop(0, CH_B // 2, pair, 0)
        plsc.subcore_barrier()
        pltpu.sync_copy(acc_f.at[pl.ds(r0, RPT)],
                        feat_o.at[c, pl.ds(r0, RPT)])
        pltpu.sync_copy(acc_c.at[pl.ds(r0, RPT)],
                        cnt_o.at[c, pl.ds(r0, RPT)])

    return k(tabs, ea, ep, z64, z8, ones8)


def _mid_body(feat_ref, cnt_ref, xp_ref, xa_ref, w0ra_ref, b0a_ref,
              w0rp_ref, b0p_ref, w1l_ref, w1r_ref, b1_ref, wlin_ref,
              blin_ref, g8_ref, base_ref):
    f32 = jnp.float32
    cp = jnp.maximum(cnt_ref[0][:, 0:1], 1.0)
    ca = jnp.maximum(cnt_ref[1][:, 0:1], 1.0)
    hp = (feat_ref[0] / cp + b0a_ref[...][None, :]
          + jnp.dot(xp_ref[...], w0ra_ref[...], preferred_element_type=f32))
    hp = jnp.where(hp >= 0, hp, 0.01 * hp)
    ha = (feat_ref[1] / ca + b0p_ref[...][None, :]
          + jnp.dot(xa_ref[...], w0rp_ref[...], preferred_element_type=f32))
    ha = jnp.where(ha >= 0, ha, 0.01 * ha)
    m1 = jnp.dot(w1l_ref[...], wlin_ref[...], preferred_element_type=f32)
    m1p = jnp.concatenate([m1, jnp.zeros_like(m1)], axis=1)
    m2 = jnp.dot(w1r_ref[...], wlin_ref[...], preferred_element_type=f32)
    bias = (jnp.dot(b1_ref[...][None, :], wlin_ref[...],
                    preferred_element_type=f32) + blin_ref[...][None, :])
    g8_ref[...] = jnp.dot(hp, m1p, preferred_element_type=f32)
    base = jnp.dot(ha, m2, preferred_element_type=f32) + bias
    base_ref[...] = jnp.concatenate(
        [base, 1.0 / ca, jnp.zeros((base.shape[0], 3), f32)], axis=1)


def _tc_mid(feat, cnt, xp, xa, w0ra, b0a, w0rp, b0p, w1l, w1r, b1,
            wlin, blin):
    return pl.pallas_call(
        _mid_body,
        grid=(GRID,),
        in_specs=[
            pl.BlockSpec((2, BM, H), lambda i: (0, i, 0)),
            pl.BlockSpec((2, BM, 8), lambda i: (0, i, 0)),
            pl.BlockSpec((BM, D), lambda i: (i, 0)),
            pl.BlockSpec((BM, D), lambda i: (i, 0)),
            pl.BlockSpec((D, H), lambda i: (0, 0)),
            pl.BlockSpec((H,), lambda i: (0,)),
            pl.BlockSpec((D, H), lambda i: (0, 0)),
            pl.BlockSpec((H,), lambda i: (0,)),
            pl.BlockSpec((H, H), lambda i: (0, 0)),
            pl.BlockSpec((H, H), lambda i: (0, 0)),
            pl.BlockSpec((H,), lambda i: (0,)),
            pl.BlockSpec((H, OUT), lambda i: (0, 0)),
            pl.BlockSpec((OUT,), lambda i: (0,)),
        ],
        out_specs=[
            pl.BlockSpec((BM, 8), lambda i: (i, 0)),
            pl.BlockSpec((BM, 8), lambda i: (i, 0)),
        ],
        out_shape=[
            jax.ShapeDtypeStruct((NA, 8), jnp.float32),
            jax.ShapeDtypeStruct((NA, 8), jnp.float32),
        ],
    )(feat, cnt, xp, xa, w0ra, b0a, w0rp, b0p, w1l, w1r, b1, wlin, blin)


def _sc_agg_l1(g8, srcs, dsts, z8):
    @functools.partial(
        pl.kernel,
        out_type=jax.ShapeDtypeStruct((NC, NR, 8), jnp.float32),
        mesh=_sc_mesh(),
        compiler_params=pltpu.CompilerParams(use_tc_tiling_on_sc=False),
        scratch_types=[
            pltpu.VMEM((CH_D, CHUNK), jnp.int32),
            pltpu.VMEM((CH_D, CHUNK), jnp.int32),
            [pltpu.VMEM((CHUNK, 8), jnp.float32)] * NBUF,
            pltpu.VMEM_SHARED((NP, 8), jnp.float32),
            pltpu.VMEM_SHARED((NR, 8), jnp.float32),
            [pltpu.SemaphoreType.DMA] * NBUF,
        ],
    )
    def k(g8_h, srcs_h, dsts_h, z8_h, out_o,
          src_v, dst_v, bufs, gtab, acc8, sems):
        c = lax.axis_index("c")
        s = lax.axis_index("s")
        w = c * NS + s
        pltpu.sync_copy(srcs_h.at[w], src_v)
        pltpu.sync_copy(dsts_h.at[w], dst_v)
        st0 = s * (NP // NS)
        pltpu.sync_copy(g8_h.at[pl.ds(st0, NP // NS)],
                        gtab.at[pl.ds(st0, NP // NS)])
        r0 = s * RPT
        pltpu.sync_copy(z8_h, acc8.at[pl.ds(r0, RPT)])
        plsc.subcore_barrier()

        for b in range(NBUF):
            pltpu.async_copy(gtab.at[src_v.at[b]], bufs[b], sems[b])

        def grp(i, carry):
            g0 = i * NBUF
            for b in range(NBUF):
                g = g0 + b
                pltpu.make_async_copy(
                    gtab.at[src_v.at[g]], bufs[b], sems[b]).wait()
                pltpu.sync_copy(bufs[b], acc8.at[dst_v.at[g]], add=True)

                @pl.when(g + NBUF < CH_D)
                def _():
                    pltpu.async_copy(
                        gtab.at[src_v.at[g + NBUF]], bufs[b], sems[b])
            return carry

        lax.fori_loop(0, CH_D // NBUF, grp, 0)
        plsc.subcore_barrier()
        pltpu.sync_copy(acc8.at[pl.ds(r0, RPT)],
                        out_o.at[c, pl.ds(r0, RPT)])

    return k(g8, srcs, dsts, z8)


def _fin_body(s8_ref, base_ref, out_ref):
    ssum = s8_ref[0] + s8_ref[1]
    out_ref[...] = (ssum[:, 0:OUT] * base_ref[:, OUT:OUT + 1]
                    + base_ref[:, 0:OUT])


def _tc_fin(s8, base):
    return pl.pallas_call(
        _fin_body,
        grid=(GRID,),
        in_specs=[
            pl.BlockSpec((2, BM, 8), lambda i: (0, i, 0)),
            pl.BlockSpec((BM, 8), lambda i: (i, 0)),
        ],
        out_specs=pl.BlockSpec((BM, OUT), lambda i: (i, 0)),
        out_shape=jax.ShapeDtypeStruct((NA, OUT), jnp.float32),
    )(s8, base)


def kernel(x_author, x_paper, edge_index_a2p, edge_index_p2a,
           W0l_a2p, b0_a2p, W0r_a2p, W0l_p2a, b0_p2a, W0r_p2a,
           W1l_a2p, b1_a2p, W1r_a2p, W1l_p2a, b1_p2a, W1r_p2a,
           Wlin, blin):
    del W1l_a2p, b1_a2p, W1r_a2p

    ea = edge_index_a2p.astype(jnp.int32).reshape(2, EROWS, CHUNK)
    ep = edge_index_p2a.astype(jnp.int32).reshape(2, EROWS, CHUNK)
    s_d, d_d = _pack_edges(edge_index_p2a, NC * NS, CH_D)

    z64 = jnp.zeros((RPT, H), jnp.float32)
    z8 = jnp.zeros((RPT, 8), jnp.float32)
    ones8 = jnp.ones((CHUNK, 8), jnp.float32)

    tabs = _make_tables(x_author, x_paper, W0l_a2p, W0l_p2a)
    feat, cnt = _sc_agg_l0(tabs, ea, ep, z64, z8, ones8)
    g8, base = _tc_mid(feat, cnt, x_paper, x_author, W0r_a2p, b0_a2p,
                       W0r_p2a, b0_p2a, W1l_p2a, W1r_p2a, b1_p2a,
                       Wlin, blin)
    s8 = _sc_agg_l1(g8, s_d, d_d, z8)
    return _tc_fin(s8, base)
